# baseline (speedup 1.0000x reference)
import functools
import math

import jax
import jax.numpy as jnp
from jax import lax
from jax.experimental import pallas as pl
from jax.experimental.pallas import tpu as pltpu
from jax.experimental.pallas import tpu_sc as plsc


_N2C1 = -2.0 * math.sqrt(2.0 / math.pi)
_N2C2 = -2.0 * 0.044715 * math.sqrt(2.0 / math.pi)

_NW = 32
_CHUNK = 16384


def _make_sc_gelu(n):
    per_w = n // _NW
    n_chunks = per_w // _CHUNK
    mesh = plsc.VectorSubcoreMesh(core_axis_name="c", subcore_axis_name="s")

    @functools.partial(
        pl.kernel,
        mesh=mesh,
        out_type=jax.ShapeDtypeStruct((n,), jnp.float32),
        scratch_types=[pltpu.VMEM((_CHUNK,), jnp.float32)],
    )
    def sc_gelu(x_hbm, o_hbm, buf):
        wid = lax.axis_index("s") * 2 + lax.axis_index("c")
        base = wid * per_w

        def chunk_body(ci, carry):
            off = base + ci * _CHUNK
            pltpu.sync_copy(x_hbm.at[pl.ds(off, _CHUNK)], buf)

            def vec_body(j, c2):
                v = buf[pl.ds(j * 16, 16)]
                nz2 = v * (_N2C1 + _N2C2 * (v * v))
                buf[pl.ds(j * 16, 16)] = v / (1.0 + jnp.exp(nz2))
                return c2

            lax.fori_loop(0, _CHUNK // 16, vec_body, 0)
            pltpu.sync_copy(buf, o_hbm.at[pl.ds(off, _CHUNK)])
            return carry

        lax.fori_loop(0, n_chunks, chunk_body, 0)

    return sc_gelu


def kernel(x, log_k_local, log_k_global):
    B, T, D = x.shape
    n = B * T * D
    y = _make_sc_gelu(n)(x.reshape(n))
    return y.reshape(B, T, D)


# speedup vs baseline: 23.1462x; 23.1462x over previous
import functools
import math

import jax
import jax.numpy as jnp
from jax.experimental import pallas as pl
from jax.experimental.pallas import tpu as pltpu


_SQRT_2_OVER_PI = math.sqrt(2.0 / math.pi)
_C2 = 0.044715 * math.sqrt(2.0 / math.pi)

_NBUF = 8
_CHUNK_ROWS = 512


def _gelu(x):
    z = x * (_SQRT_2_OVER_PI + _C2 * (x * x))
    t = jnp.tanh(z)
    h = 0.5 * x
    return h + h * t


def _pipelined_body(n_chunks, x_hbm, o_hbm, in_buf, out_buf, in_sem, out_sem):
    cr = _CHUNK_ROWS

    def start_in(i, b):
        pltpu.make_async_copy(
            x_hbm.at[pl.ds(i * cr, cr)], in_buf.at[b], in_sem.at[b]
        ).start()

    for k in range(_NBUF):
        start_in(k, k)

    def loop_body(i, carry):
        b = jax.lax.rem(i, _NBUF)
        pltpu.make_async_copy(
            x_hbm.at[pl.ds(i * cr, cr)], in_buf.at[b], in_sem.at[b]
        ).wait()

        @pl.when(i >= _NBUF)
        def _():
            pltpu.make_async_copy(
                out_buf.at[b], o_hbm.at[pl.ds((i - _NBUF) * cr, cr)], out_sem.at[b]
            ).wait()

        out_buf[b] = _gelu(in_buf[b])
        pltpu.make_async_copy(
            out_buf.at[b], o_hbm.at[pl.ds(i * cr, cr)], out_sem.at[b]
        ).start()

        @pl.when(i + _NBUF < n_chunks)
        def _():
            start_in(i + _NBUF, b)

        return carry

    jax.lax.fori_loop(0, n_chunks, loop_body, 0)

    for k in range(_NBUF):
        i = n_chunks - _NBUF + k
        pltpu.make_async_copy(
            out_buf.at[i % _NBUF], o_hbm.at[pl.ds(i * cr, cr)], out_sem.at[i % _NBUF]
        ).wait()


def kernel(x, log_k_local, log_k_global):
    B, T, D = x.shape
    rows = B * T
    n_chunks = rows // _CHUNK_ROWS
    x2 = x.reshape(rows, D)
    y = pl.pallas_call(
        functools.partial(_pipelined_body, n_chunks),
        in_specs=[pl.BlockSpec(memory_space=pltpu.MemorySpace.HBM)],
        out_specs=pl.BlockSpec(memory_space=pltpu.MemorySpace.HBM),
        out_shape=jax.ShapeDtypeStruct((rows, D), x.dtype),
        scratch_shapes=[
            pltpu.VMEM((_NBUF, _CHUNK_ROWS, D), x.dtype),
            pltpu.VMEM((_NBUF, _CHUNK_ROWS, D), x.dtype),
            pltpu.SemaphoreType.DMA((_NBUF,)),
            pltpu.SemaphoreType.DMA((_NBUF,)),
        ],
    )(x2)
    return y.reshape(B, T, D)


# speedup vs baseline: 23.1508x; 1.0002x over previous
import functools
import math

import jax
import jax.numpy as jnp
from jax.experimental import pallas as pl
from jax.experimental.pallas import tpu as pltpu


_SQRT_2_OVER_PI = math.sqrt(2.0 / math.pi)
_C2 = 0.044715 * math.sqrt(2.0 / math.pi)

_NBUF = 8
_CHUNK_ROWS = 256


def _gelu(x):
    z = x * (_SQRT_2_OVER_PI + _C2 * (x * x))
    t = jnp.tanh(z)
    h = 0.5 * x
    return h + h * t


def _pipelined_body(n_chunks, x_hbm, o_hbm, in_buf, out_buf, in_sem, out_sem):
    cr = _CHUNK_ROWS

    def start_in(i, b):
        pltpu.make_async_copy(
            x_hbm.at[pl.ds(i * cr, cr)], in_buf.at[b], in_sem.at[b]
        ).start()

    for k in range(_NBUF):
        start_in(k, k)

    def loop_body(i, carry):
        b = jax.lax.rem(i, _NBUF)
        pltpu.make_async_copy(
            x_hbm.at[pl.ds(i * cr, cr)], in_buf.at[b], in_sem.at[b]
        ).wait()

        @pl.when(i >= _NBUF)
        def _():
            pltpu.make_async_copy(
                out_buf.at[b], o_hbm.at[pl.ds((i - _NBUF) * cr, cr)], out_sem.at[b]
            ).wait()

        out_buf[b] = _gelu(in_buf[b])
        pltpu.make_async_copy(
            out_buf.at[b], o_hbm.at[pl.ds(i * cr, cr)], out_sem.at[b]
        ).start()

        @pl.when(i + _NBUF < n_chunks)
        def _():
            start_in(i + _NBUF, b)

        return carry

    jax.lax.fori_loop(0, n_chunks, loop_body, 0)

    for k in range(_NBUF):
        i = n_chunks - _NBUF + k
        pltpu.make_async_copy(
            out_buf.at[i % _NBUF], o_hbm.at[pl.ds(i * cr, cr)], out_sem.at[i % _NBUF]
        ).wait()


def kernel(x, log_k_local, log_k_global):
    B, T, D = x.shape
    rows = B * T
    n_chunks = rows // _CHUNK_ROWS
    x2 = x.reshape(rows, D)
    y = pl.pallas_call(
        functools.partial(_pipelined_body, n_chunks),
        in_specs=[pl.BlockSpec(memory_space=pltpu.MemorySpace.HBM)],
        out_specs=pl.BlockSpec(memory_space=pltpu.MemorySpace.HBM),
        out_shape=jax.ShapeDtypeStruct((rows, D), x.dtype),
        scratch_shapes=[
            pltpu.VMEM((_NBUF, _CHUNK_ROWS, D), x.dtype),
            pltpu.VMEM((_NBUF, _CHUNK_ROWS, D), x.dtype),
            pltpu.SemaphoreType.DMA((_NBUF,)),
            pltpu.SemaphoreType.DMA((_NBUF,)),
        ],
    )(x2)
    return y.reshape(B, T, D)
